# baseline (speedup 1.0000x reference)
import functools

import jax
import jax.numpy as jnp
from jax import lax
from jax.experimental import pallas as pl
from jax.experimental.pallas import tpu as pltpu
from jax.experimental.pallas import tpu_sc as plsc

_NC, _NS, _L = 2, 16, 16
_NW = _NC * _NS


def _sc_add(x_flat, pe_flat, B, S, D):
    N = B * S * D
    HALF = N // _NC
    CE = 524288
    NCHUNK = HALF // CE

    mesh = plsc.VectorSubcoreMesh(core_axis_name="c", subcore_axis_name="s")

    @functools.partial(
        pl.kernel,
        out_type=jax.ShapeDtypeStruct((N,), jnp.float32),
        mesh=mesh,
        scratch_types=[
            pltpu.VMEM_SHARED((CE,), jnp.float32),
            pltpu.VMEM_SHARED((CE,), jnp.float32),
            pltpu.SemaphoreType.DMA,
            pltpu.SemaphoreType.DMA,
            pltpu.SemaphoreType.DMA,
            pltpu.SemaphoreType.DMA,
        ],
    )
    def k(x_hbm, pe_hbm, out_hbm, s0, s1, si0, si1, so0, so1):
        cid = lax.axis_index("c")
        sid = lax.axis_index("s")
        sb = [s0, s1]
        si = [si0, si1]
        so = [so0, so1]

        @pl.when(sid == 0)
        def _():
            base = cid * HALF

            def start_in(i):
                return pltpu.async_copy(
                    x_hbm.at[pl.ds(base + i * CE, CE)], sb[i % 2], si[i % 2])

            in_d = [None, None]
            out_d = [None, None]
            in_d[0] = start_in(0)
            for i in range(NCHUNK):
                c = i % 2
                if i + 1 < NCHUNK:
                    if out_d[(i + 1) % 2] is not None:
                        out_d[(i + 1) % 2].wait()
                    in_d[(i + 1) % 2] = start_in(i + 1)
                in_d[c].wait()
                out_d[c] = pltpu.async_copy(
                    sb[c], out_hbm.at[pl.ds(base + i * CE, CE)], so[c])
            out_d[0].wait()
            out_d[1].wait()

    return k(x_flat, pe_flat)


def kernel(x, pe):
    B, S, D = x.shape
    out = _sc_add(x.reshape(B * S * D), pe.reshape(S * D), B, S, D)
    return out.reshape(B, S, D)


# speedup vs baseline: 1.1090x; 1.1090x over previous
import functools

import jax
import jax.numpy as jnp
from jax import lax
from jax.experimental import pallas as pl
from jax.experimental.pallas import tpu as pltpu
from jax.experimental.pallas import tpu_sc as plsc

_NC, _NS, _L = 2, 16, 16
_NW = _NC * _NS


def _tc_body(x_ref, pe_ref, o_ref):
    o_ref[...] = x_ref[...] + pe_ref[...]


def _tc_add(x, pe, s_hi):
    B, S, D = x.shape
    BS = 512
    return pl.pallas_call(
        _tc_body,
        grid=(s_hi // BS, B),
        in_specs=[
            pl.BlockSpec((1, BS, D), lambda s, b: (b, s, 0)),
            pl.BlockSpec((BS, D), lambda s, b: (s, 0)),
        ],
        out_specs=pl.BlockSpec((1, BS, D), lambda s, b: (b, s, 0)),
        out_shape=jax.ShapeDtypeStruct((B, s_hi, D), x.dtype),
    )(x, pe)


def _sc_add(x_flat, pe_flat, B, S, D, s0, s_len):
    PE_PER_W = s_len // _NW
    CH = PE_PER_W if PE_PER_W <= 16 else 16
    NCH = PE_PER_W // CH
    CHE = CH * D
    NT = NCH * B

    mesh = plsc.VectorSubcoreMesh(core_axis_name="c", subcore_axis_name="s")

    @functools.partial(
        pl.kernel,
        out_type=jax.ShapeDtypeStruct((B * s_len * D,), jnp.float32),
        mesh=mesh,
        scratch_types=[
            pltpu.VMEM((CHE,), jnp.float32),
            pltpu.VMEM((CHE,), jnp.float32),
            pltpu.VMEM((CHE,), jnp.float32),
            pltpu.VMEM((CHE,), jnp.float32),
            pltpu.SemaphoreType.DMA,
            pltpu.SemaphoreType.DMA,
            pltpu.SemaphoreType.DMA,
            pltpu.SemaphoreType.DMA,
            pltpu.SemaphoreType.DMA,
            pltpu.SemaphoreType.DMA,
        ],
    )
    def k(x_hbm, pe_hbm, out_hbm, bx0, bx1, bp0, bp1,
          sx0, sx1, sp0, sp1, so0, so1):
        bx, bp = [bx0, bx1], [bp0, bp1]
        sx, sp, so = [sx0, sx1], [sp0, sp1], [so0, so1]
        wid = lax.axis_index("s") * _NC + lax.axis_index("c")
        w_s0 = s0 + wid * PE_PER_W
        w_o0 = wid * PE_PER_W

        def xoff(t):
            p, b = divmod(t, B)
            return b * S * D + (w_s0 + p * CH) * D

        def ooff(t):
            p, b = divmod(t, B)
            return b * s_len * D + (w_o0 + p * CH) * D

        def start_xin(t):
            return pltpu.async_copy(
                x_hbm.at[pl.ds(xoff(t), CHE)], bx[t % 2], sx[t % 2])

        def start_pin(p):
            return pltpu.async_copy(
                pe_hbm.at[pl.ds((w_s0 + p * CH) * D, CHE)], bp[p % 2], sp[p % 2])

        pin_d = start_pin(0)
        xin_d = [None, None]
        out_d = [None, None]
        xin_d[0] = start_xin(0)

        for t in range(NT):
            c = t % 2
            p, b = divmod(t, B)
            if t + 1 < NT:
                if out_d[(t + 1) % 2] is not None:
                    out_d[(t + 1) % 2].wait()
                xin_d[(t + 1) % 2] = start_xin(t + 1)
            xin_d[c].wait()
            if b == 0:
                pin_d.wait()
                if p + 1 < NCH:
                    pin_d = start_pin(p + 1)
            cur_bp = bp[p % 2]
            cur_bx = bx[c]

            @plsc.parallel_loop(0, CHE // _L, unroll=8)
            def _vec(i):
                plsc.addupdate(cur_bx.at[pl.ds(i * _L, _L)],
                               cur_bp[pl.ds(i * _L, _L)])

            out_d[c] = pltpu.async_copy(
                cur_bx, out_hbm.at[pl.ds(ooff(t), CHE)], so[c])

        out_d[0].wait()
        out_d[1].wait()

    return k(x_flat, pe_flat)


def kernel(x, pe):
    B, S, D = x.shape
    S_TC = 3584
    out_tc = _tc_add(x, pe, S_TC)
    out_sc = _sc_add(x.reshape(B * S * D), pe.reshape(S * D),
                     B, S, D, S_TC, S - S_TC)
    return jnp.concatenate(
        [out_tc, out_sc.reshape(B, S - S_TC, D)], axis=1)


# speedup vs baseline: 1.9864x; 1.7911x over previous
import functools

import jax
import jax.numpy as jnp
from jax import lax
from jax.experimental import pallas as pl
from jax.experimental.pallas import tpu as pltpu
from jax.experimental.pallas import tpu_sc as plsc

_NC, _NS, _L = 2, 16, 16
_NW = _NC * _NS


def _tc_body(x_ref, pe_ref, o_ref):
    o_ref[...] = x_ref[...] + pe_ref[...]


def _tc_add(x, pe, s_hi, BS):
    B, S, D = x.shape
    return pl.pallas_call(
        _tc_body,
        grid=(s_hi // BS, B),
        in_specs=[
            pl.BlockSpec((1, BS, D), lambda s, b: (b, s, 0)),
            pl.BlockSpec((BS, D), lambda s, b: (s, 0)),
        ],
        out_specs=pl.BlockSpec((1, BS, D), lambda s, b: (b, s, 0)),
        out_shape=jax.ShapeDtypeStruct((B, s_hi, D), x.dtype),
    )(x, pe)


def _sc_add(x, pe, s0, s_len):
    B, S, D = x.shape
    PE_PER_W = s_len // _NW
    CH = min(PE_PER_W, 16)
    NCH = PE_PER_W // CH
    NT = NCH * B
    NV = D // _L

    mesh = plsc.VectorSubcoreMesh(core_axis_name="c", subcore_axis_name="s")

    @functools.partial(
        pl.kernel,
        out_type=jax.ShapeDtypeStruct((B, s_len, D), jnp.float32),
        mesh=mesh,
        scratch_types=[
            pltpu.VMEM((CH, D), jnp.float32),
            pltpu.VMEM((CH, D), jnp.float32),
            pltpu.VMEM((CH, D), jnp.float32),
            pltpu.VMEM((CH, D), jnp.float32),
            pltpu.SemaphoreType.DMA,
            pltpu.SemaphoreType.DMA,
            pltpu.SemaphoreType.DMA,
            pltpu.SemaphoreType.DMA,
            pltpu.SemaphoreType.DMA,
            pltpu.SemaphoreType.DMA,
        ],
    )
    def k(x_hbm, pe_hbm, out_hbm, bx0, bx1, bp0, bp1,
          sx0, sx1, sp0, sp1, so0, so1):
        bx, bp = [bx0, bx1], [bp0, bp1]
        sx, sp, so = [sx0, sx1], [sp0, sp1], [so0, so1]
        wid = lax.axis_index("s") * _NC + lax.axis_index("c")
        w_s0 = s0 + wid * PE_PER_W
        w_o0 = wid * PE_PER_W

        def start_xin(t):
            p, b = divmod(t, B)
            return pltpu.async_copy(
                x_hbm.at[b, pl.ds(w_s0 + p * CH, CH)], bx[t % 2], sx[t % 2])

        def start_pin(p):
            return pltpu.async_copy(
                pe_hbm.at[pl.ds(w_s0 + p * CH, CH)], bp[p % 2], sp[p % 2])

        pin_d = start_pin(0)
        xin_d = [None, None]
        out_d = [None, None]
        xin_d[0] = start_xin(0)

        for t in range(NT):
            c = t % 2
            p, b = divmod(t, B)
            if t + 1 < NT:
                if out_d[(t + 1) % 2] is not None:
                    out_d[(t + 1) % 2].wait()
                xin_d[(t + 1) % 2] = start_xin(t + 1)
            xin_d[c].wait()
            if b == 0:
                pin_d.wait()
                if p + 1 < NCH:
                    pin_d = start_pin(p + 1)
            cur_bp = bp[p % 2]
            cur_bx = bx[c]

            @plsc.parallel_loop(0, CH)
            def _row(r):
                @plsc.parallel_loop(0, NV, unroll=8)
                def _vec(j):
                    plsc.addupdate(cur_bx.at[r, pl.ds(j * _L, _L)],
                                   cur_bp[r, pl.ds(j * _L, _L)])

            out_d[c] = pltpu.async_copy(
                cur_bx, out_hbm.at[b, pl.ds(w_o0 + p * CH, CH)], so[c])

        out_d[0].wait()
        out_d[1].wait()

    return k(x, pe)


def kernel(x, pe):
    B, S, D = x.shape
    S_TC = 3072
    out_tc = _tc_add(x, pe, S_TC, 1536)
    out_sc = _sc_add(x, pe, S_TC, S - S_TC)
    return jnp.concatenate([out_tc, out_sc], axis=1)


# speedup vs baseline: 4.4673x; 2.2490x over previous
import jax
import jax.numpy as jnp
from jax.experimental import pallas as pl


def _add_body(x_ref, pe_ref, o_ref):
    o_ref[...] = x_ref[...] + pe_ref[...]


def kernel(x, pe):
    B, S, D = x.shape
    BS = 2048
    return pl.pallas_call(
        _add_body,
        grid=(S // BS, B),
        in_specs=[
            pl.BlockSpec((1, BS, D), lambda s, b: (b, s, 0)),
            pl.BlockSpec((BS, D), lambda s, b: (s, 0)),
        ],
        out_specs=pl.BlockSpec((1, BS, D), lambda s, b: (b, s, 0)),
        out_shape=jax.ShapeDtypeStruct(x.shape, x.dtype),
    )(x, pe)
